# Initial kernel scaffold; baseline (speedup 1.0000x reference)
import functools

import jax
import jax.numpy as jnp
from jax import lax
from jax.experimental import pallas as pl
from jax.experimental.pallas import tpu as pltpu
from jax.experimental.pallas import tpu_sc as plsc

N = 100000
D = 512
G = 64
L = 16
NT = 32
ACC_W = D + L
ROWS_PER_CHUNK = 32
N_CHUNKS = N // ROWS_PER_CHUNK
BASE_CHUNKS = N_CHUNKS // NT
EXTRA_CHUNKS = N_CHUNKS % NT
MAX_CHUNKS = BASE_CHUNKS + 1
MAX_ROWS = MAX_CHUNKS * ROWS_PER_CHUNK


def _seg_body(z_hbm, batch_hbm, out_hbm, batch_v, zbuf, acc, sem0, sem1):
    cid = lax.axis_index("c")
    sid = lax.axis_index("s")
    wid = sid * 2 + cid
    n_chunks = BASE_CHUNKS + jnp.where(wid < EXTRA_CHUNKS, 1, 0)
    chunk0 = wid * BASE_CHUNKS + jnp.minimum(wid, EXTRA_CHUNKS)
    row0 = chunk0 * ROWS_PER_CHUNK

    zero16 = jnp.zeros((L,), jnp.float32)

    def _zero_row(i, carry):
        for j in range(ACC_W // L):
            acc[i, pl.ds(j * L, L)] = zero16
        return carry

    lax.fori_loop(0, G + 1, _zero_row, 0)

    bstart = jnp.minimum(row0, N - MAX_ROWS)
    boff = row0 - bstart
    pltpu.sync_copy(batch_hbm.at[pl.ds(bstart, MAX_ROWS)], batch_v)

    def _dma(j, buf):
        src = jnp.where(j < n_chunks,
                        (chunk0 + j) * ROWS_PER_CHUNK,
                        N - ROWS_PER_CHUNK)
        sem = sem0 if buf == 0 else sem1
        return pltpu.make_async_copy(
            z_hbm.at[pl.ds(src, ROWS_PER_CHUNK)], zbuf.at[buf], sem)

    def _process(i, buf):
        valid = i < n_chunks
        loff = jnp.where(valid, boff + i * ROWS_PER_CHUNK, 0)
        vvec = jnp.broadcast_to(valid, (L,))
        for sub in range(ROWS_PER_CHUNK // L):
            rbase = sub * L
            segs = batch_v[pl.ds(loff + rbase, L)]
            segs = jnp.where(vvec, segs, jnp.full((L,), G, jnp.int32))
            smin = jnp.min(segs)
            smax = jnp.max(segs)

            @pl.when(smin == smax)
            def _fast():
                def _cols(c8, carry):
                    for cc in range(8):
                        cs = pl.ds((c8 * 8 + cc) * L, L)
                        v0 = zbuf[buf, rbase + 0, cs] + zbuf[buf, rbase + 1, cs]
                        v1 = zbuf[buf, rbase + 2, cs] + zbuf[buf, rbase + 3, cs]
                        v2 = zbuf[buf, rbase + 4, cs] + zbuf[buf, rbase + 5, cs]
                        v3 = zbuf[buf, rbase + 6, cs] + zbuf[buf, rbase + 7, cs]
                        v4 = zbuf[buf, rbase + 8, cs] + zbuf[buf, rbase + 9, cs]
                        v5 = zbuf[buf, rbase + 10, cs] + zbuf[buf, rbase + 11, cs]
                        v6 = zbuf[buf, rbase + 12, cs] + zbuf[buf, rbase + 13, cs]
                        v7 = zbuf[buf, rbase + 14, cs] + zbuf[buf, rbase + 15, cs]
                        v = ((v0 + v1) + (v2 + v3)) + ((v4 + v5) + (v6 + v7))
                        acc[smin, cs] += v
                    return carry

                lax.fori_loop(0, D // L // 8, _cols, 0)
                acc[smin, pl.ds(D, L)] += jnp.full((L,), 16.0, jnp.float32)

            @pl.when(smin != smax)
            def _slow():
                iota = lax.iota(jnp.int32, L)

                def _row(r, carry):
                    s_r = jnp.max(jnp.where(iota == r, segs,
                                            jnp.zeros((L,), jnp.int32)))

                    def _cols(c8, c2):
                        for cc in range(8):
                            cs = pl.ds((c8 * 8 + cc) * L, L)
                            acc[s_r, cs] += zbuf[buf, rbase + r, cs]
                        return c2

                    lax.fori_loop(0, D // L // 8, _cols, 0)
                    acc[s_r, pl.ds(D, L)] += jnp.full((L,), 1.0, jnp.float32)
                    return carry

                lax.fori_loop(0, L, _row, 0)

    _dma(0, 0).start()
    _dma(1, 1).start()

    def _pair(p, carry):
        i0 = 2 * p
        _dma(i0, 0).wait()
        _process(i0, 0)

        @pl.when(i0 + 2 < MAX_CHUNKS)
        def _():
            _dma(i0 + 2, 0).start()

        _dma(i0 + 1, 1).wait()
        _process(i0 + 1, 1)

        @pl.when(i0 + 3 < MAX_CHUNKS)
        def _():
            _dma(i0 + 3, 1).start()

        return carry

    lax.fori_loop(0, MAX_CHUNKS // 2, _pair, 0)

    pltpu.sync_copy(acc.at[pl.ds(0, G)], out_hbm.at[wid])


_seg_kernel = functools.partial(
    pl.kernel,
    out_type=jax.ShapeDtypeStruct((NT, G, ACC_W), jnp.float32),
    mesh=plsc.VectorSubcoreMesh(core_axis_name="c", subcore_axis_name="s"),
    scratch_types=[
        pltpu.VMEM((MAX_ROWS,), jnp.int32),
        pltpu.VMEM((2, ROWS_PER_CHUNK, D), jnp.float32),
        pltpu.VMEM((G + 1, ACC_W), jnp.float32),
        pltpu.SemaphoreType.DMA,
        pltpu.SemaphoreType.DMA,
    ],
)(_seg_body)


def _mlp_body(p_ref, w1_ref, b1_ref, w2_ref, b2_ref, o_ref):
    total = jnp.sum(p_ref[...], axis=0)
    sums = total[:, :D]
    cnt = total[:, D:D + 1]
    mean = sums / jnp.maximum(cnt, 1.0)
    h = lax.dot_general(mean, w1_ref[...], (((1,), (1,)), ((), ())),
                        preferred_element_type=jnp.float32) + b1_ref[...]
    h = jnp.maximum(h, 0.0)
    y = lax.dot_general(h, w2_ref[...], (((1,), (1,)), ((), ())),
                        preferred_element_type=jnp.float32) + b2_ref[...]
    o_ref[...] = jax.nn.sigmoid(y)


def kernel(z, batch, W1, b1, W2, b2):
    partials = _seg_kernel(z, batch.astype(jnp.int32))
    return pl.pallas_call(
        _mlp_body,
        out_shape=jax.ShapeDtypeStruct((G, 1), jnp.float32),
    )(partials, W1, b1.reshape(1, G), W2, b2.reshape(1, 1))


# speedup vs baseline: 4.3016x; 4.3016x over previous
import functools

import jax
import jax.numpy as jnp
from jax import lax
from jax.experimental import pallas as pl
from jax.experimental.pallas import tpu as pltpu
from jax.experimental.pallas import tpu_sc as plsc

N = 100000
D = 512
G = 64
L = 16
NT = 32
ACC_W = D + L
ROWS_PER_CHUNK = 32
N_CHUNKS = N // ROWS_PER_CHUNK
BASE_CHUNKS = N_CHUNKS // NT
EXTRA_CHUNKS = N_CHUNKS % NT
MAX_CHUNKS = BASE_CHUNKS + 1
MAX_ROWS = MAX_CHUNKS * ROWS_PER_CHUNK


def _seg_body(z_hbm, batch_hbm, out_hbm, batch_v, zbuf, acc, sem0, sem1):
    cid = lax.axis_index("c")
    sid = lax.axis_index("s")
    wid = sid * 2 + cid
    n_chunks = BASE_CHUNKS + jnp.where(wid < EXTRA_CHUNKS, 1, 0)
    chunk0 = wid * BASE_CHUNKS + jnp.minimum(wid, EXTRA_CHUNKS)
    row0 = chunk0 * ROWS_PER_CHUNK

    zero16 = jnp.zeros((L,), jnp.float32)

    def _zero_row(i, carry):
        for j in range(ACC_W // L):
            acc[i, pl.ds(j * L, L)] = zero16
        return carry

    lax.fori_loop(0, G + 1, _zero_row, 0)

    bstart = jnp.minimum(row0, N - MAX_ROWS)
    boff = row0 - bstart
    pltpu.sync_copy(batch_hbm.at[pl.ds(bstart, MAX_ROWS)],
                    batch_v.at[pl.ds(0, MAX_ROWS)])

    def _dma(j, buf):
        src = jnp.where(j < n_chunks,
                        (chunk0 + j) * ROWS_PER_CHUNK,
                        N - ROWS_PER_CHUNK)
        sem = sem0 if buf == 0 else sem1
        return pltpu.make_async_copy(
            z_hbm.at[pl.ds(src, ROWS_PER_CHUNK)], zbuf.at[buf], sem)

    def _process(i, buf):
        valid = i < n_chunks
        loff = jnp.where(valid, boff + i * ROWS_PER_CHUNK, 0)
        for sub in range(ROWS_PER_CHUNK // L):
            rbase = sub * L
            segs = batch_v[pl.ds(loff + rbase, L)]
            smin = jnp.where(valid, segs[0], G)
            smax = jnp.where(valid, segs[L - 1], G)

            @pl.when(smin == smax)
            def _fast():
                def _cols(c8, carry):
                    for cc in range(8):
                        cs = pl.ds((c8 * 8 + cc) * L, L)
                        v0 = zbuf[buf, rbase + 0, cs] + zbuf[buf, rbase + 1, cs]
                        v1 = zbuf[buf, rbase + 2, cs] + zbuf[buf, rbase + 3, cs]
                        v2 = zbuf[buf, rbase + 4, cs] + zbuf[buf, rbase + 5, cs]
                        v3 = zbuf[buf, rbase + 6, cs] + zbuf[buf, rbase + 7, cs]
                        v4 = zbuf[buf, rbase + 8, cs] + zbuf[buf, rbase + 9, cs]
                        v5 = zbuf[buf, rbase + 10, cs] + zbuf[buf, rbase + 11, cs]
                        v6 = zbuf[buf, rbase + 12, cs] + zbuf[buf, rbase + 13, cs]
                        v7 = zbuf[buf, rbase + 14, cs] + zbuf[buf, rbase + 15, cs]
                        v = ((v0 + v1) + (v2 + v3)) + ((v4 + v5) + (v6 + v7))
                        acc[smin, cs] += v
                    return carry

                lax.fori_loop(0, D // L // 8, _cols, 0)
                acc[smin, pl.ds(D, L)] += jnp.full((L,), 16.0, jnp.float32)

            @pl.when(smin != smax)
            def _slow():
                def _row(r, carry):
                    s_r = batch_v[pl.ds(loff + rbase + r, L)][0]

                    def _cols(c8, c2):
                        for cc in range(8):
                            cs = pl.ds((c8 * 8 + cc) * L, L)
                            acc[s_r, cs] += zbuf[buf, rbase + r, cs]
                        return c2

                    lax.fori_loop(0, D // L // 8, _cols, 0)
                    acc[s_r, pl.ds(D, L)] += jnp.full((L,), 1.0, jnp.float32)
                    return carry

                lax.fori_loop(0, L, _row, 0)

    _dma(0, 0).start()
    _dma(1, 1).start()

    def _pair(p, carry):
        i0 = 2 * p
        _dma(i0, 0).wait()
        _process(i0, 0)

        @pl.when(i0 + 2 < MAX_CHUNKS)
        def _():
            _dma(i0 + 2, 0).start()

        _dma(i0 + 1, 1).wait()
        _process(i0 + 1, 1)

        @pl.when(i0 + 3 < MAX_CHUNKS)
        def _():
            _dma(i0 + 3, 1).start()

        return carry

    lax.fori_loop(0, MAX_CHUNKS // 2, _pair, 0)

    pltpu.sync_copy(acc.at[pl.ds(0, G)], out_hbm.at[wid])


_seg_kernel = functools.partial(
    pl.kernel,
    out_type=jax.ShapeDtypeStruct((NT, G, ACC_W), jnp.float32),
    mesh=plsc.VectorSubcoreMesh(core_axis_name="c", subcore_axis_name="s"),
    scratch_types=[
        pltpu.VMEM((MAX_ROWS + L,), jnp.int32),
        pltpu.VMEM((2, ROWS_PER_CHUNK, D), jnp.float32),
        pltpu.VMEM((G + 1, ACC_W), jnp.float32),
        pltpu.SemaphoreType.DMA,
        pltpu.SemaphoreType.DMA,
    ],
)(_seg_body)


def _mlp_body(p_ref, w1_ref, b1_ref, w2p_ref, b2p_ref, o_ref):
    total = jnp.sum(p_ref[...], axis=0)
    sums = total[:, :D]
    cnt = total[:, D:D + 1]
    mean = sums / jnp.maximum(cnt, 1.0)
    h = lax.dot_general(mean, w1_ref[...], (((1,), (1,)), ((), ())),
                        preferred_element_type=jnp.float32) + b1_ref[...]
    h = jnp.maximum(h, 0.0)
    y = lax.dot_general(h, w2p_ref[...], (((1,), (0,)), ((), ())),
                        preferred_element_type=jnp.float32) + b2p_ref[...]
    o_ref[...] = jax.nn.sigmoid(y)


def kernel(z, batch, W1, b1, W2, b2):
    partials = _seg_kernel(z, batch.astype(jnp.int32))
    w2p = jnp.pad(W2.T, ((0, 0), (0, 127)))
    b2p = jnp.broadcast_to(b2.reshape(1, 1), (1, 128))
    out = pl.pallas_call(
        _mlp_body,
        out_shape=jax.ShapeDtypeStruct((G, 128), jnp.float32),
    )(partials, W1, b1.reshape(1, G), w2p, b2p)
    return out[:, :1]
